# baseline (speedup 1.0000x reference)
import functools

import jax
import jax.numpy as jnp
from jax.experimental import pallas as pl

B, N, D, H, O, TOPK = 1024, 50, 512, 512, 512, 16
BBLK = 64


def _bf16(x):
    return x.astype(jnp.bfloat16)


def _dot_t(a, b):
    return jax.lax.dot_general(_bf16(a), _bf16(b), (((1,), (1,)), ((), ())),
                               preferred_element_type=jnp.float32)


def _fused_kernel(c_ref, e_ref, wq_ref, bq_ref, wk_ref, bk_ref, wv_ref,
                  bv_ref, wo_ref, bo_ref, o_ref):
    q = _dot_t(c_ref[...], wq_ref[...]) + bq_ref[...]
    qb = _bf16(q).astype(jnp.float32) * (H ** -0.5)
    cols = []
    for n in range(N):
        kn = _dot_t(e_ref[:, n, :], wk_ref[...]) + bk_ref[...]
        knb = _bf16(kn).astype(jnp.float32)
        cols.append(jnp.sum(qb * knb, axis=-1, keepdims=True))
    logits = jnp.concatenate(cols, axis=1)
    m = jnp.max(logits, axis=-1, keepdims=True)
    p = jnp.exp(logits - m)
    s = p / jnp.sum(p, axis=-1, keepdims=True)
    lane = jax.lax.broadcasted_iota(jnp.int32, (1, N), 1)
    rank_cols = []
    for n in range(N):
        sn = s[:, n:n + 1]
        beats = (s > sn) | ((s == sn) & (lane < n))
        rank_cols.append(jnp.sum(beats.astype(jnp.int32), axis=-1,
                                 keepdims=True))
    rank = jnp.concatenate(rank_cols, axis=1)
    w = jnp.where(rank < TOPK, s, 0.0)
    s_tot = jnp.sum(w, axis=-1, keepdims=True)
    e_w = w[:, 0:1] * e_ref[:, 0, :]
    for n in range(1, N):
        e_w = e_w + w[:, n:n + 1] * e_ref[:, n, :]
    ws = _dot_t(e_w, wv_ref[...]) + s_tot * bv_ref[...]
    o_ref[...] = _dot_t(ws, wo_ref[...]) + bo_ref[...]


@functools.partial(jax.jit, static_argnames=("interpret",))
def _run(class_embedding, entity_embeddings, Wq, bq, Wk, bk, Wv, bv, Wo, bo,
         interpret=False):
    grid = (B // BBLK,)
    full = lambda *shape: pl.BlockSpec(shape, lambda i: (0,) * len(shape))
    return pl.pallas_call(
        _fused_kernel,
        grid=grid,
        in_specs=[
            pl.BlockSpec((BBLK, D), lambda i: (i, 0)),
            pl.BlockSpec((BBLK, N, D), lambda i: (i, 0, 0)),
            full(H, D),
            full(1, H),
            full(H, D),
            full(1, H),
            full(H, D),
            full(1, H),
            full(O, H),
            full(1, O),
        ],
        out_specs=pl.BlockSpec((BBLK, O), lambda i: (i, 0)),
        out_shape=jax.ShapeDtypeStruct((B, O), jnp.float32),
        interpret=interpret,
    )(class_embedding, entity_embeddings, Wq, bq.reshape(1, H), Wk,
      bk.reshape(1, H), Wv, bv.reshape(1, H), Wo, bo.reshape(1, O))


def kernel(class_embedding, entity_embeddings, Wq, bq, Wk, bk, Wv, bv, Wo, bo):
    return _run(class_embedding, entity_embeddings, Wq, bq, Wk, bk, Wv, bv,
                Wo, bo)
